# baseline (speedup 1.0000x reference)
import functools

import jax
import jax.numpy as jnp
from jax import lax
from jax.experimental import pallas as pl
from jax.experimental.pallas import tpu as pltpu
from jax.experimental.pallas import tpu_sc as plsc

L = 16
NTILES = 32
GB = 128


def _pack32(v):
    r, c = v.shape
    return lax.bitcast_convert_type(v.reshape(r, c // 2, 2), jnp.int32)


def _unpack32(v):
    r, cw = v.shape
    return lax.bitcast_convert_type(v, jnp.bfloat16).reshape(r, 2 * cw)


def _k1_body(x_ref, wa_ref, wb_ref, b1_ref, a_ref, b_ref):
    x = x_ref[...]
    a_ref[...] = jnp.dot(x, wa_ref[...], preferred_element_type=jnp.float32) + b1_ref[...]
    b_ref[...] = jnp.dot(x, wb_ref[...], preferred_element_type=jnp.float32)


def _node_tables(x, W1, b1):
    n, d = x.shape
    wa = (W1[:, :d] - W1[:, d:]).T
    wb = W1[:, d:].T
    br = 1000
    grid = (n // br,)
    a, b = pl.pallas_call(
        _k1_body,
        grid=grid,
        in_specs=[
            pl.BlockSpec((br, d), lambda i: (i, 0)),
            pl.BlockSpec((d, d), lambda i: (0, 0)),
            pl.BlockSpec((d, d), lambda i: (0, 0)),
            pl.BlockSpec((1, d), lambda i: (0, 0)),
        ],
        out_specs=[
            pl.BlockSpec((br, d), lambda i: (i, 0)),
            pl.BlockSpec((br, d), lambda i: (i, 0)),
        ],
        out_shape=[
            jax.ShapeDtypeStruct((n, d), jnp.float32),
            jax.ShapeDtypeStruct((n, d), jnp.float32),
        ],
    )(x, wa, wb, b1.reshape(1, d))
    return a, b


GB2 = 128


def _k2_body(e, d, n, rows_pt, a_hbm, b_hbm, dst_hbm, src_hbm, g_hbm,
             idxd, idxs, bufa, bufb, bufo, semg, semw):
    sub = lax.axis_index("s")
    wid = sub * 2 + lax.axis_index("c")
    tile_base = wid * rows_pt
    nch = (rows_pt + GB2 - 1) // GB2
    hpr = d // (2 * L)

    def base_of(c):
        return tile_base + jnp.minimum(c * GB2, rows_pt - GB2)

    def stage(c, s):
        base = base_of(c)
        pltpu.sync_copy(dst_hbm.at[pl.ds(base, GB2)], idxd[s])
        pltpu.sync_copy(src_hbm.at[pl.ds(base, GB2)], idxs[s])
        pltpu.async_copy(a_hbm.at[idxd[s]], bufa[s], semg[s])
        pltpu.async_copy(b_hbm.at[idxs[s]], bufb[s], semg[s])

    def run(c, s):
        @pl.when(c + 1 < nch)
        def _():
            stage(c + 1, 1 - s)

        pltpu.make_async_copy(a_hbm.at[pl.ds(0, GB2)], bufa[s], semg[s]).wait()
        pltpu.make_async_copy(b_hbm.at[pl.ds(0, GB2)], bufb[s], semg[s]).wait()

        @pl.when(c >= 2)
        def _():
            pltpu.make_async_copy(bufo[s], g_hbm.at[pl.ds(0, GB2)], semw[s]).wait()

        def row(r, _):
            for v in range(hpr):
                alo = bufa[s][r, pl.ds(2 * v * L, L)]
                ahi = bufa[s][r, pl.ds((2 * v + 1) * L, L)]
                blo = bufb[s][r, pl.ds(2 * v * L, L)]
                bhi = bufb[s][r, pl.ds((2 * v + 1) * L, L)]
                olo = jnp.maximum(alo + blo, 0.0)
                ohi = jnp.maximum(ahi + bhi, 0.0)
                packed = plsc.pack(olo, ohi, format=plsc.PackFormat.INTERLEAVED)
                bufo[s][r, pl.ds(v * L, L)] = plsc.bitcast(packed, jnp.int32)
            return 0

        lax.fori_loop(0, GB2, row, 0, unroll=2)
        pltpu.async_copy(bufo[s], g_hbm.at[pl.ds(base_of(c), GB2)], semw[s])

    stage(0, 0)

    def chunk(i, _):
        c = i * 2
        run(c, 0)

        @pl.when(c + 1 < nch)
        def _():
            run(c + 1, 1)
        return 0

    lax.fori_loop(0, (nch + 1) // 2, chunk, 0)
    for s in range(2):
        if nch > s:
            pltpu.make_async_copy(bufo[s], g_hbm.at[pl.ds(0, GB2)], semw[s]).wait()


def _gather_add_relu(a, b, dst, src):
    n, d = a.shape
    e = dst.shape[0]
    rows_pt = e // NTILES
    mesh = plsc.VectorSubcoreMesh(core_axis_name="c", subcore_axis_name="s")
    f = pl.kernel(
        functools.partial(_k2_body, e, d, n, rows_pt),
        out_type=jax.ShapeDtypeStruct((e, d // 2), jnp.int32),
        mesh=mesh,
        compiler_params=pltpu.CompilerParams(needs_layout_passes=False),
        scratch_types=[
            [pltpu.VMEM((GB2,), jnp.int32)] * 2,
            [pltpu.VMEM((GB2,), jnp.int32)] * 2,
            [pltpu.VMEM((GB2, d), jnp.float32)] * 2,
            [pltpu.VMEM((GB2, d), jnp.float32)] * 2,
            [pltpu.VMEM((GB2, d // 2), jnp.int32)] * 2,
            [pltpu.SemaphoreType.DMA] * 2,
            [pltpu.SemaphoreType.DMA] * 2,
        ],
    )
    return f(a, b, dst, src)


def _k3_body(g_ref, w_ref, b2_ref, h_ref):
    h_ref[...] = (
        jnp.dot(g_ref[...], w_ref[...], preferred_element_type=jnp.float32)
        + b2_ref[...]
    )


def _edge_matmul(g32, W2, b2):
    g = _unpack32(g32)
    e, d = g.shape
    import numpy as np
    perm = np.concatenate([
        32 * blk + 16 * (np.arange(32) % 2) + np.arange(32) // 2
        for blk in range(d // 32)
    ])
    w = W2.T[perm].astype(jnp.bfloat16)
    br = 1280
    grid = (e // br,)
    return pl.pallas_call(
        _k3_body,
        grid=grid,
        in_specs=[
            pl.BlockSpec((br, d), lambda i: (i, 0)),
            pl.BlockSpec((d, d), lambda i: (0, 0)),
            pl.BlockSpec((1, d), lambda i: (0, 0)),
        ],
        out_specs=pl.BlockSpec((br, d), lambda i: (i, 0)),
        out_shape=jax.ShapeDtypeStruct((e, d), jnp.float32),
    )(g, w, b2.reshape(1, d))


NPT = 320
CH = 8000
MCAP = 8192


FIRE = 4


def _k4_body(e, d, h_hbm, dst_hbm, agg_hbm,
             dstc, midx, dval, hbuf, aggb, semb):
    wid = lax.axis_index("s") * 2 + lax.axis_index("c")
    lo = wid * NPT
    vpr = d // L
    nchunk = e // CH
    iota = lax.iota(jnp.int32, L)

    def initv(i, _):
        aggb[pl.ds(i * L, L)] = jnp.full((L,), -jnp.inf, jnp.float32)
        return 0
    lax.fori_loop(0, (NPT + 1) * d // L, initv, 0, unroll=4)

    def initm(i, _):
        midx[pl.ds(i * L, L)] = i * L + iota
        return 0
    lax.fori_loop(0, MCAP // L, initm, 0, unroll=4)

    def rmw(f):
        fb = f * GB

        def edge(j, _):
            djv = plsc.load_gather(dval, [jnp.zeros((L,), jnp.int32) + (fb + j)])
            ldv = djv - lo
            valid = (ldv >= 0) & (ldv < NPT)
            row = jnp.where(valid, ldv, NPT)
            base = row * d
            idxs = [base + (iota + v * L) for v in range(vpr)]
            olds = [plsc.load_gather(aggb, [idxs[v]]) for v in range(vpr)]
            hvs = [hbuf[fb + j, pl.ds(v * L, L)] for v in range(vpr)]
            for v in range(vpr):
                plsc.store_scatter(aggb, [idxs[v]], jnp.maximum(olds[v], hvs[v]))
            return 0

        lax.fori_loop(0, GB, edge, 0)

    def chunk(c, _):
        pltpu.sync_copy(dst_hbm.at[pl.ds(c * CH, CH)], dstc)

        def scan(v, cnt):
            dv = dstc[pl.ds(v * L, L)]
            ld = dv - lo
            m = (ld >= 0) & (ld < NPT)
            pc = plsc.all_reduce_population_count(m)
            pos = plsc.cumsum(m.astype(jnp.int32)) + cnt - 1
            eidx = (c * CH + v * L) + iota
            plsc.store_scatter(midx, [pos], eidx, mask=m)
            return cnt + pc

        cntv = lax.fori_loop(0, CH // L, scan, jnp.zeros((L,), jnp.int32),
                             unroll=4)
        cnt = jnp.max(cntv)
        nb = (cnt + GB - 1) // GB

        def rnd(r, _):
            r0 = r * FIRE
            for f in range(FIRE):
                @pl.when(r0 + f < nb)
                def _(f=f):
                    mslice = midx.at[pl.ds((r0 + f) * GB, GB)]
                    pltpu.async_copy(h_hbm.at[mslice],
                                     hbuf.at[pl.ds(f * GB, GB)], semb)
                    pltpu.async_copy(dst_hbm.at[mslice],
                                     dval.at[pl.ds(f * GB, GB)], semb)
            for f in range(FIRE):
                @pl.when(r0 + f < nb)
                def _(f=f):
                    pltpu.make_async_copy(
                        h_hbm.at[pl.ds(0, GB)],
                        hbuf.at[pl.ds(f * GB, GB)], semb).wait()
                    pltpu.make_async_copy(
                        dst_hbm.at[pl.ds(0, GB)],
                        dval.at[pl.ds(f * GB, GB)], semb).wait()
                    rmw(f)
            return 0

        lax.fori_loop(0, (nb + FIRE - 1) // FIRE, rnd, 0)
        return 0

    lax.fori_loop(0, nchunk, chunk, 0)
    pltpu.sync_copy(aggb.at[pl.ds(0, NPT * d)],
                    agg_hbm.at[pl.ds(wid * NPT * d, NPT * d)])


def _segment_max(h, dst):
    e, d = h.shape
    mesh = plsc.VectorSubcoreMesh(core_axis_name="c", subcore_axis_name="s")
    f = pl.kernel(
        functools.partial(_k4_body, e, d),
        out_type=jax.ShapeDtypeStruct((NTILES * NPT * d,), jnp.float32),
        mesh=mesh,
        compiler_params=pltpu.CompilerParams(needs_layout_passes=False),
        scratch_types=[
            pltpu.VMEM((CH,), jnp.int32),
            pltpu.VMEM((MCAP,), jnp.int32),
            pltpu.VMEM((FIRE * GB,), jnp.int32),
            pltpu.VMEM((FIRE * GB, d), jnp.float32),
            pltpu.VMEM(((NPT + 1) * d,), jnp.float32),
            pltpu.SemaphoreType.DMA,
        ],
    )
    return f(h, dst).reshape(NTILES * NPT, d)


def _k5_body(n, agg_ref, g_ref, b_ref, y_ref):
    a = agg_ref[...]
    a = jnp.where(a == -jnp.inf, 0.0, a)
    s = jnp.sum(a, axis=0, keepdims=True)
    sq = jnp.sum(a * a, axis=0, keepdims=True)
    mean = s / n
    var = sq / n - mean * mean
    y = (a[:n] - mean) / jnp.sqrt(var + 1e-5) * g_ref[...] + b_ref[...]
    y_ref[...] = y


def _batchnorm(agg, n, gamma, beta):
    npad, d = agg.shape
    return pl.pallas_call(
        functools.partial(_k5_body, n),
        out_shape=jax.ShapeDtypeStruct((n, d), jnp.float32),
    )(agg, gamma.reshape(1, d), beta.reshape(1, d))


def kernel(x, edge_index, edge_attr, W1, b1, W2, b2, gamma, beta):
    src = edge_index[0]
    dst = edge_index[1]
    a, b = _node_tables(x, W1, b1)
    g = _gather_add_relu(a, b, dst, src)
    h = _edge_matmul(g, W2, b2)
    agg = _segment_max(h, dst)
    y = _batchnorm(agg, x.shape[0], gamma, beta)
    return (y, edge_index, edge_attr)


# speedup vs baseline: 1.4353x; 1.4353x over previous
import functools

import jax
import jax.numpy as jnp
from jax import lax
from jax.experimental import pallas as pl
from jax.experimental.pallas import tpu as pltpu
from jax.experimental.pallas import tpu_sc as plsc

L = 16
NTILES = 32
GB = 128


def _pack32(v):
    r, c = v.shape
    return lax.bitcast_convert_type(v.reshape(r, c // 2, 2), jnp.int32)


def _unpack32(v):
    r, cw = v.shape
    return lax.bitcast_convert_type(v, jnp.bfloat16).reshape(r, 2 * cw)


def _k1_body(x_ref, wa_ref, wb_ref, b1_ref, a_ref, b_ref):
    x = x_ref[...]
    a_ref[...] = jnp.dot(x, wa_ref[...], preferred_element_type=jnp.float32) + b1_ref[...]
    b_ref[...] = jnp.dot(x, wb_ref[...], preferred_element_type=jnp.float32)


def _node_tables(x, W1, b1):
    n, d = x.shape
    wa = (W1[:, :d] - W1[:, d:]).T
    wb = W1[:, d:].T
    br = 1000
    grid = (n // br,)
    a, b = pl.pallas_call(
        _k1_body,
        grid=grid,
        in_specs=[
            pl.BlockSpec((br, d), lambda i: (i, 0)),
            pl.BlockSpec((d, d), lambda i: (0, 0)),
            pl.BlockSpec((d, d), lambda i: (0, 0)),
            pl.BlockSpec((1, d), lambda i: (0, 0)),
        ],
        out_specs=[
            pl.BlockSpec((br, d), lambda i: (i, 0)),
            pl.BlockSpec((br, d), lambda i: (i, 0)),
        ],
        out_shape=[
            jax.ShapeDtypeStruct((n, d), jnp.float32),
            jax.ShapeDtypeStruct((n, d), jnp.float32),
        ],
    )(x, wa, wb, b1.reshape(1, d))
    return a, b


GB2 = 128


def _k2_body(e, d, n, rows_pt, a_hbm, b_hbm, dst_hbm, src_hbm, g_hbm,
             idxd, idxs, bufa, bufb, bufo, semg, semw):
    sub = lax.axis_index("s")
    wid = sub * 2 + lax.axis_index("c")
    tile_base = wid * rows_pt
    nch = (rows_pt + GB2 - 1) // GB2

    def base_of(c):
        return tile_base + jnp.minimum(c * GB2, rows_pt - GB2)

    def stage(c, s):
        base = base_of(c)
        pltpu.sync_copy(dst_hbm.at[pl.ds(base, GB2)], idxd[s])
        pltpu.sync_copy(src_hbm.at[pl.ds(base, GB2)], idxs[s])
        pltpu.async_copy(a_hbm.at[idxd[s]], bufa[s], semg[s])
        pltpu.async_copy(b_hbm.at[idxs[s]], bufb[s], semg[s])

    def run(c, s):
        @pl.when(c + 1 < nch)
        def _():
            stage(c + 1, 1 - s)

        pltpu.make_async_copy(a_hbm.at[pl.ds(0, GB2)], bufa[s], semg[s]).wait()
        pltpu.make_async_copy(b_hbm.at[pl.ds(0, GB2)], bufb[s], semg[s]).wait()

        @pl.when(c >= 2)
        def _():
            pltpu.make_async_copy(bufo[s], g_hbm.at[pl.ds(0, GB2)], semw[s]).wait()

        def row(r, _):
            for v in range(d // L):
                av = bufa[s][r, pl.ds(v * L, L)]
                bv = bufb[s][r, pl.ds(v * L, L)]
                bufo[s][r, pl.ds(v * L, L)] = jnp.maximum(av + bv, 0.0)
            return 0

        lax.fori_loop(0, GB2, row, 0, unroll=2)
        pltpu.async_copy(bufo[s], g_hbm.at[pl.ds(base_of(c), GB2)], semw[s])

    stage(0, 0)

    def chunk(i, _):
        c = i * 2
        run(c, 0)

        @pl.when(c + 1 < nch)
        def _():
            run(c + 1, 1)
        return 0

    lax.fori_loop(0, (nch + 1) // 2, chunk, 0)
    for s in range(2):
        if nch > s:
            pltpu.make_async_copy(bufo[s], g_hbm.at[pl.ds(0, GB2)], semw[s]).wait()


def _gather_add_relu(a, b, dst, src):
    n, d = a.shape
    e = dst.shape[0]
    rows_pt = e // NTILES
    mesh = plsc.VectorSubcoreMesh(core_axis_name="c", subcore_axis_name="s")
    f = pl.kernel(
        functools.partial(_k2_body, e, d, n, rows_pt),
        out_type=jax.ShapeDtypeStruct((e, d), jnp.float32),
        mesh=mesh,
        compiler_params=pltpu.CompilerParams(needs_layout_passes=False),
        scratch_types=[
            [pltpu.VMEM((GB2,), jnp.int32)] * 2,
            [pltpu.VMEM((GB2,), jnp.int32)] * 2,
            [pltpu.VMEM((GB2, d), jnp.float32)] * 2,
            [pltpu.VMEM((GB2, d), jnp.float32)] * 2,
            [pltpu.VMEM((GB2, d), jnp.float32)] * 2,
            [pltpu.SemaphoreType.DMA] * 2,
            [pltpu.SemaphoreType.DMA] * 2,
        ],
    )
    return f(a, b, dst, src)


def _k3_body(g_ref, w_ref, b2_ref, h_ref):
    h_ref[...] = (
        jnp.dot(g_ref[...], w_ref[...], preferred_element_type=jnp.float32)
        + b2_ref[...]
    )


def _edge_matmul(g, W2, b2):
    e, d = g.shape
    br = 1280
    grid = (e // br,)
    return pl.pallas_call(
        _k3_body,
        grid=grid,
        in_specs=[
            pl.BlockSpec((br, d), lambda i: (i, 0)),
            pl.BlockSpec((d, d), lambda i: (0, 0)),
            pl.BlockSpec((1, d), lambda i: (0, 0)),
        ],
        out_specs=pl.BlockSpec((br, d), lambda i: (i, 0)),
        out_shape=jax.ShapeDtypeStruct((e, d), jnp.float32),
    )(g, W2.T, b2.reshape(1, d))


NPT = 320
CH = 8000
MCAP = 8192


FIRE = 4


def _k4_body(e, d, h_hbm, dst_hbm, agg_hbm,
             dstc, midx, dval, hbuf, aggb, semb):
    wid = lax.axis_index("s") * 2 + lax.axis_index("c")
    lo = wid * NPT
    vpr = d // L
    nchunk = e // CH
    iota = lax.iota(jnp.int32, L)

    def initv(i, _):
        aggb[pl.ds(i * L, L)] = jnp.full((L,), -jnp.inf, jnp.float32)
        return 0
    lax.fori_loop(0, (NPT + 1) * d // L, initv, 0, unroll=4)

    def initm(i, _):
        midx[pl.ds(i * L, L)] = i * L + iota
        return 0
    lax.fori_loop(0, MCAP // L, initm, 0, unroll=4)

    def rmw(f, nedge):
        fb = f * GB

        def edge(j, _):
            djv = plsc.load_gather(dval, [jnp.zeros((L,), jnp.int32) + (fb + j)])
            ldv = djv - lo
            valid = (ldv >= 0) & (ldv < NPT)
            row = jnp.where(valid, ldv, NPT)
            base = row * d
            idxs = [base + (iota + v * L) for v in range(vpr)]
            olds = [plsc.load_gather(aggb, [idxs[v]]) for v in range(vpr)]
            hvs = [hbuf[fb + j, pl.ds(v * L, L)] for v in range(vpr)]
            for v in range(vpr):
                plsc.store_scatter(aggb, [idxs[v]], jnp.maximum(olds[v], hvs[v]))
            return 0

        lax.fori_loop(0, nedge, edge, 0)

    def chunk(c, _):
        pltpu.sync_copy(dst_hbm.at[pl.ds(c * CH, CH)], dstc)

        def scan(v, cnt):
            dv = dstc[pl.ds(v * L, L)]
            ld = dv - lo
            m = (ld >= 0) & (ld < NPT)
            pc = plsc.all_reduce_population_count(m)
            pos = plsc.cumsum(m.astype(jnp.int32)) + cnt - 1
            eidx = (c * CH + v * L) + iota
            plsc.store_scatter(midx, [pos], eidx, mask=m)
            return cnt + pc

        cntv = lax.fori_loop(0, CH // L, scan, jnp.zeros((L,), jnp.int32),
                             unroll=4)
        cnt = jnp.max(cntv)
        nb = (cnt + GB - 1) // GB

        def rnd(r, _):
            r0 = r * FIRE
            for f in range(FIRE):
                @pl.when(r0 + f < nb)
                def _(f=f):
                    mslice = midx.at[pl.ds((r0 + f) * GB, GB)]
                    pltpu.async_copy(h_hbm.at[mslice],
                                     hbuf.at[pl.ds(f * GB, GB)], semb)
                    pltpu.async_copy(dst_hbm.at[mslice],
                                     dval.at[pl.ds(f * GB, GB)], semb)
            for f in range(FIRE):
                @pl.when(r0 + f < nb)
                def _(f=f):
                    pltpu.make_async_copy(
                        h_hbm.at[pl.ds(0, GB)],
                        hbuf.at[pl.ds(f * GB, GB)], semb).wait()
                    pltpu.make_async_copy(
                        dst_hbm.at[pl.ds(0, GB)],
                        dval.at[pl.ds(f * GB, GB)], semb).wait()
                    rmw(f, jnp.minimum(GB, cnt - (r0 + f) * GB))
            return 0

        lax.fori_loop(0, (nb + FIRE - 1) // FIRE, rnd, 0)
        return 0

    lax.fori_loop(0, nchunk, chunk, 0)
    pltpu.sync_copy(aggb.at[pl.ds(0, NPT * d)],
                    agg_hbm.at[pl.ds(wid * NPT * d, NPT * d)])


def _segment_max(h, dst):
    e, d = h.shape
    mesh = plsc.VectorSubcoreMesh(core_axis_name="c", subcore_axis_name="s")
    f = pl.kernel(
        functools.partial(_k4_body, e, d),
        out_type=jax.ShapeDtypeStruct((NTILES * NPT * d,), jnp.float32),
        mesh=mesh,
        compiler_params=pltpu.CompilerParams(needs_layout_passes=False),
        scratch_types=[
            pltpu.VMEM((CH,), jnp.int32),
            pltpu.VMEM((MCAP,), jnp.int32),
            pltpu.VMEM((FIRE * GB,), jnp.int32),
            pltpu.VMEM((FIRE * GB, d), jnp.float32),
            pltpu.VMEM(((NPT + 1) * d,), jnp.float32),
            pltpu.SemaphoreType.DMA,
        ],
    )
    return f(h, dst).reshape(NTILES * NPT, d)


def _k5_body(n, agg_ref, g_ref, b_ref, y_ref):
    a = agg_ref[...]
    a = jnp.where(a == -jnp.inf, 0.0, a)
    s = jnp.sum(a, axis=0, keepdims=True)
    sq = jnp.sum(a * a, axis=0, keepdims=True)
    mean = s / n
    var = sq / n - mean * mean
    y = (a[:n] - mean) / jnp.sqrt(var + 1e-5) * g_ref[...] + b_ref[...]
    y_ref[...] = y


def _batchnorm(agg, n, gamma, beta):
    npad, d = agg.shape
    return pl.pallas_call(
        functools.partial(_k5_body, n),
        out_shape=jax.ShapeDtypeStruct((n, d), jnp.float32),
    )(agg, gamma.reshape(1, d), beta.reshape(1, d))


def kernel(x, edge_index, edge_attr, W1, b1, W2, b2, gamma, beta):
    src = edge_index[0]
    dst = edge_index[1]
    a, b = _node_tables(x, W1, b1)
    g = _gather_add_relu(a, b, dst, src)
    h = _edge_matmul(g, W2, b2)
    agg = _segment_max(h, dst)
    y = _batchnorm(agg, x.shape[0], gamma, beta)
    return (y, edge_index, edge_attr)
